# baseline (speedup 1.0000x reference)
import functools

import jax
import jax.numpy as jnp
from jax import lax
from jax.experimental import pallas as pl
from jax.experimental.pallas import tpu as pltpu
from jax.experimental.pallas import tpu_sc as plsc

NN = 512
OPS = 32
E = NN * NN // 2
NSLOT = NN * NN
PAD = 16384
EP = E + PAD
NC, NS, L = 2, 16, 16
NW = NC * NS
S = NSLOT // NW
DPW = S // NN
CH = 8192
GB = 128
TB = 2048
EPT = EP // NW
CH2 = EPT // 4


def _tc_sample_body(a_ref, u_ref, w_ref, l_ref):
    i = pl.program_id(0)
    a = a_ref[...]
    u = jnp.clip(u_ref[...], 1e-6, 1.0 - 1e-6)
    w = u / (u + (1.0 - u) * jnp.exp(-a))
    is_pad = i >= E // TB
    w_ref[...] = jnp.where(is_pad, 0.0, w)
    l_ref[...] = jnp.where(is_pad, 0.0, a)


def _sample(a_t, u_t):
    last = E // TB - 1
    return pl.pallas_call(
        _tc_sample_body,
        grid=(EP // TB,),
        in_specs=[pl.BlockSpec((OPS, TB), lambda i: (0, jnp.minimum(i, last)))] * 2,
        out_specs=[pl.BlockSpec((OPS, TB), lambda i: (0, i))] * 2,
        out_shape=[jax.ShapeDtypeStruct((OPS, EP), jnp.float32)] * 2,
    )(a_t, u_t)


def _sc_transpose_body(wt_hbm, at_hbm, we_hbm, ae_hbm, tin, tout, sem):
    wid = lax.axis_index("s") * NC + lax.axis_index("c")
    iota = lax.broadcasted_iota(jnp.int32, (L,), 0)
    iota32 = iota * OPS
    ccw = CH2 // GB

    for src_hbm, dst_hbm in ((wt_hbm, we_hbm), (at_hbm, ae_hbm)):
        for c in range(EPT // CH2):
            e0 = wid * EPT + c * CH2
            pltpu.sync_copy(src_hbm.at[:, pl.ds(e0 // GB, ccw), :, :], tin)

            def tr_body(i, carry):
                cc = i >> 3
                ec = i & 7
                for rb in range(4):
                    for r_in in range(8):
                        v = tin[rb, cc, r_in, pl.ds(ec * L, L)]
                        plsc.store_scatter(
                            tout,
                            [iota32 + ((cc * GB + ec * L) * OPS
                                       + rb * 8 + r_in)], v)
                return carry

            lax.fori_loop(0, CH2 // L, tr_body, 0)
            pltpu.sync_copy(tout, dst_hbm.at[pl.ds(e0 * OPS, CH2 * OPS)])


_sc_transpose = functools.partial(
    pl.kernel,
    out_type=(jax.ShapeDtypeStruct((EP * OPS,), jnp.float32),
              jax.ShapeDtypeStruct((EP * OPS,), jnp.float32)),
    mesh=plsc.VectorSubcoreMesh(core_axis_name="c", subcore_axis_name="s"),
    compiler_params=pltpu.CompilerParams(
        needs_layout_passes=False, use_tc_tiling_on_sc=False,
        disable_bounds_checks=True),
    scratch_types=[
        pltpu.VMEM((4, CH2 // GB, 8, GB), jnp.float32),
        pltpu.VMEM((CH2 * OPS,), jnp.float32),
        pltpu.SemaphoreType.DMA,
    ],
)(_sc_transpose_body)


def _sc_body(i0_hbm, i1_hbm, opw_hbm, alph_hbm, out_hbm,
             win, i0b0, i1b0, i0b1, i1b1, rows0, rows1, tb0, tb1,
             csem0, csem1, gsem0, gsem1, osem0, osem1):
    wid = lax.axis_index("s") * NC + lax.axis_index("c")
    base = wid * S
    dst0 = wid * DPW
    iota = lax.broadcasted_iota(jnp.int32, (L,), 0)

    def init_row(j, carry):
        for k in range(GB // L):
            sent = E + ((j * GB + k * L + iota) & (PAD - 1))
            win[j, pl.ds(k * L, L)] = sent
        return carry

    lax.fori_loop(0, S // GB, init_row, 0)

    i0b = (i0b0, i0b1)
    i1b = (i1b0, i1b1)
    csem = (csem0, csem1)
    UNROLL = 8
    NCHK = E // CH

    def issue_chunk(c, b):
        return (pltpu.async_copy(i0_hbm.at[pl.ds(c * CH, CH)], i0b[b], csem[b]),
                pltpu.async_copy(i1_hbm.at[pl.ds(c * CH, CH)], i1b[b], csem[b]))

    pend = issue_chunk(0, 0)
    for c in range(NCHK):
        b = c & 1
        cur = pend
        if c + 1 < NCHK:
            pend = issue_chunk(c + 1, 1 - b)
        cur[0].wait()
        cur[1].wait()

        def scan_body(i, carry, c=c, b=b):
            for k in range(UNROLL):
                off = i * (UNROLL * L) + k * L
                v0 = i0b[b][pl.ds(off, L)]
                v1 = i1b[b][pl.ds(off, L)]
                rel = v0 * NN + v1 - base
                m = (rel >= 0) & (rel < S)
                relc = jnp.where(m, rel, 0)
                evec = (c * CH) + off + iota
                plsc.store_scatter(
                    win, [relc >> 7, relc & (GB - 1)], evec, mask=m)
            return carry

        lax.fori_loop(0, CH // (UNROLL * L), scan_body, 0)

    rows = (rows0, rows1)
    tb = (tb0, tb1)
    gsem = (gsem0, gsem1)
    osem = (osem0, osem1)
    tvec0 = (iota >> 3) * (4 * 1024) + (iota & 7) * GB
    tvec1 = ((iota + L) >> 3) * (4 * 1024) + ((iota + L) & 7) * GB

    def issue_gather(src_hbm, d, b):
        return tuple(
            pltpu.async_copy(src_hbm.at[win.at[d * 4 + q]],
                             rows[b].at[pl.ds(q * GB, GB)], gsem[b])
            for q in range(4))

    units = [(p, s, d) for p, s in ((0, opw_hbm), (1, alph_hbm))
             for d in range(DPW)]
    wr = [None, None]
    gp = issue_gather(units[0][1], units[0][2], 0)
    for u, (plane, src_hbm, d) in enumerate(units):
        b = u & 1
        cur = gp
        if u + 1 < len(units):
            nxt = units[u + 1]
            gp = issue_gather(nxt[1], nxt[2], 1 - b)
        for dsc in cur:
            dsc.wait()
        if wr[b] is not None:
            wr[b].wait()

        def tr_body(s, carry, b=b):
            soff = (s >> 7) * 1024 + (s & (GB - 1))
            plsc.store_scatter(tb[b], [tvec0 + soff],
                               rows[b][s, pl.ds(0, L)])
            plsc.store_scatter(tb[b], [tvec1 + soff],
                               rows[b][s, pl.ds(L, L)])
            return carry

        lax.fori_loop(0, NN, tr_body, 0)
        wr[b] = pltpu.async_copy(tb[b], out_hbm.at[plane, dst0 + d],
                                 osem[b])
    wr[0].wait()
    wr[1].wait()


_sc_scatter = functools.partial(
    pl.kernel,
    out_type=jax.ShapeDtypeStruct((2, NN, OPS * NN), jnp.float32),
    mesh=plsc.VectorSubcoreMesh(core_axis_name="c", subcore_axis_name="s"),
    compiler_params=pltpu.CompilerParams(
        needs_layout_passes=False, use_tc_tiling_on_sc=False,
        disable_bounds_checks=True),
    scratch_types=[
        pltpu.VMEM((S // GB, GB), jnp.int32),
        pltpu.VMEM((CH,), jnp.int32),
        pltpu.VMEM((CH,), jnp.int32),
        pltpu.VMEM((CH,), jnp.int32),
        pltpu.VMEM((CH,), jnp.int32),
        pltpu.VMEM((NN, OPS), jnp.float32),
        pltpu.VMEM((NN, OPS), jnp.float32),
        pltpu.VMEM((OPS * NN,), jnp.float32),
        pltpu.VMEM((OPS * NN,), jnp.float32),
        pltpu.SemaphoreType.DMA,
        pltpu.SemaphoreType.DMA,
        pltpu.SemaphoreType.DMA,
        pltpu.SemaphoreType.DMA,
        pltpu.SemaphoreType.DMA,
        pltpu.SemaphoreType.DMA,
    ],
)(_sc_body)


def kernel(alphas, noise_u, idx):
    idx = idx.astype(jnp.int32)
    w_t, a_t = _sample(alphas.T, noise_u.T)
    w_t4 = w_t.reshape(4, 8, EP // GB, GB).transpose(0, 2, 1, 3)
    a_t4 = a_t.reshape(4, 8, EP // GB, GB).transpose(0, 2, 1, 3)
    w_e, a_e = _sc_transpose(w_t4, a_t4)
    out = _sc_scatter(idx[0], idx[1],
                      w_e.reshape(EP, OPS), a_e.reshape(EP, OPS))
    out6 = out.reshape(2, NN, 4, 4, 8, GB)
    return out6.transpose(0, 1, 3, 5, 2, 4).reshape(2, NN, NN, OPS)


# speedup vs baseline: 1.0346x; 1.0346x over previous
import functools

import jax
import jax.numpy as jnp
from jax import lax
from jax.experimental import pallas as pl
from jax.experimental.pallas import tpu as pltpu
from jax.experimental.pallas import tpu_sc as plsc

NN = 512
OPS = 32
E = NN * NN // 2
NSLOT = NN * NN
PAD = 16384
EP = E + PAD
NC, NS, L = 2, 16, 16
NW = NC * NS
S = NSLOT // NW
DPW = S // NN
CH = 8192
GB = 128
TB = 2048
EPT = EP // NW
CH2 = EPT // 12


def _tc_sample_body(a_ref, u_ref, w_ref, l_ref):
    i = pl.program_id(0)
    a = a_ref[...]
    u = jnp.clip(u_ref[...], 1e-6, 1.0 - 1e-6)
    w = u / (u + (1.0 - u) * jnp.exp(-a))
    is_pad = i >= E // TB
    w_ref[...] = jnp.where(is_pad, 0.0, w)
    l_ref[...] = jnp.where(is_pad, 0.0, a)


def _sample(a_t, u_t):
    last = E // TB - 1
    return pl.pallas_call(
        _tc_sample_body,
        grid=(EP // TB,),
        in_specs=[pl.BlockSpec((OPS, TB), lambda i: (0, jnp.minimum(i, last)))] * 2,
        out_specs=[pl.BlockSpec((OPS, TB), lambda i: (0, i))] * 2,
        out_shape=[jax.ShapeDtypeStruct((OPS, EP), jnp.float32)] * 2,
    )(a_t, u_t)


def _sc_transpose_body(wt_hbm, at_hbm, we_hbm, ae_hbm,
                       tin0, tin1, tout0, tout1,
                       isem0, isem1, osem0, osem1):
    wid = lax.axis_index("s") * NC + lax.axis_index("c")
    iota = lax.broadcasted_iota(jnp.int32, (L,), 0)
    iota32 = iota * OPS
    ccw = CH2 // GB
    tin = (tin0, tin1)
    tout = (tout0, tout1)
    isem = (isem0, isem1)
    osem = (osem0, osem1)

    units = [(src, dst, c)
             for src, dst in ((wt_hbm, we_hbm), (at_hbm, ae_hbm))
             for c in range(EPT // CH2)]

    def issue_load(u, b):
        src, _, c = units[u]
        e0 = wid * EPT + c * CH2
        return pltpu.async_copy(src.at[:, pl.ds(e0 // GB, ccw), :, :],
                                tin[b], isem[b])

    wr = [None, None]
    pend = issue_load(0, 0)
    for u in range(len(units)):
        b = u & 1
        cur = pend
        if u + 1 < len(units):
            pend = issue_load(u + 1, 1 - b)
        cur.wait()
        if wr[b] is not None:
            wr[b].wait()

        def tr_body(i, carry, b=b):
            cc = i >> 3
            ec = i & 7
            for rb in range(4):
                for r_in in range(8):
                    v = tin[b][rb, cc, r_in, pl.ds(ec * L, L)]
                    plsc.store_scatter(
                        tout[b],
                        [iota32 + ((cc * GB + ec * L) * OPS
                                   + rb * 8 + r_in)], v)
            return carry

        lax.fori_loop(0, CH2 // L, tr_body, 0)
        _, dst, c = units[u]
        e0 = wid * EPT + c * CH2
        wr[b] = pltpu.async_copy(tout[b], dst.at[pl.ds(e0 * OPS, CH2 * OPS)],
                                 osem[b])
    wr[0].wait()
    wr[1].wait()


_sc_transpose = functools.partial(
    pl.kernel,
    out_type=(jax.ShapeDtypeStruct((EP * OPS,), jnp.float32),
              jax.ShapeDtypeStruct((EP * OPS,), jnp.float32)),
    mesh=plsc.VectorSubcoreMesh(core_axis_name="c", subcore_axis_name="s"),
    compiler_params=pltpu.CompilerParams(
        needs_layout_passes=False, use_tc_tiling_on_sc=False,
        disable_bounds_checks=True),
    scratch_types=[
        pltpu.VMEM((4, CH2 // GB, 8, GB), jnp.float32),
        pltpu.VMEM((4, CH2 // GB, 8, GB), jnp.float32),
        pltpu.VMEM((CH2 * OPS,), jnp.float32),
        pltpu.VMEM((CH2 * OPS,), jnp.float32),
        pltpu.SemaphoreType.DMA,
        pltpu.SemaphoreType.DMA,
        pltpu.SemaphoreType.DMA,
        pltpu.SemaphoreType.DMA,
    ],
)(_sc_transpose_body)


def _sc_body(i0_hbm, i1_hbm, opw_hbm, alph_hbm, out_hbm,
             win, i0b0, i1b0, i0b1, i1b1, rows0, rows1, tb0, tb1,
             csem0, csem1, gsem0, gsem1, osem0, osem1):
    wid = lax.axis_index("s") * NC + lax.axis_index("c")
    base = wid * S
    dst0 = wid * DPW
    iota = lax.broadcasted_iota(jnp.int32, (L,), 0)

    def init_row(j, carry):
        for k in range(GB // L):
            sent = E + ((j * GB + k * L + iota) & (PAD - 1))
            win[j, pl.ds(k * L, L)] = sent
        return carry

    lax.fori_loop(0, S // GB, init_row, 0)

    i0b = (i0b0, i0b1)
    i1b = (i1b0, i1b1)
    csem = (csem0, csem1)
    UNROLL = 8
    NCHK = E // CH

    def issue_chunk(c, b):
        return (pltpu.async_copy(i0_hbm.at[pl.ds(c * CH, CH)], i0b[b], csem[b]),
                pltpu.async_copy(i1_hbm.at[pl.ds(c * CH, CH)], i1b[b], csem[b]))

    pend = issue_chunk(0, 0)
    for c in range(NCHK):
        b = c & 1
        cur = pend
        if c + 1 < NCHK:
            pend = issue_chunk(c + 1, 1 - b)
        cur[0].wait()
        cur[1].wait()

        def scan_body(i, carry, c=c, b=b):
            for k in range(UNROLL):
                off = i * (UNROLL * L) + k * L
                v0 = i0b[b][pl.ds(off, L)]
                v1 = i1b[b][pl.ds(off, L)]
                rel = v0 * NN + v1 - base
                m = (rel >= 0) & (rel < S)
                relc = jnp.where(m, rel, 0)
                evec = (c * CH) + off + iota
                plsc.store_scatter(
                    win, [relc >> 7, relc & (GB - 1)], evec, mask=m)
            return carry

        lax.fori_loop(0, CH // (UNROLL * L), scan_body, 0)

    rows = (rows0, rows1)
    tb = (tb0, tb1)
    gsem = (gsem0, gsem1)
    osem = (osem0, osem1)
    tvec0 = (iota >> 3) * (4 * 1024) + (iota & 7) * GB
    tvec1 = ((iota + L) >> 3) * (4 * 1024) + ((iota + L) & 7) * GB

    def issue_gather(src_hbm, d, b):
        return tuple(
            pltpu.async_copy(src_hbm.at[win.at[d * 4 + q]],
                             rows[b].at[pl.ds(q * GB, GB)], gsem[b])
            for q in range(4))

    units = [(p, s, d) for p, s in ((0, opw_hbm), (1, alph_hbm))
             for d in range(DPW)]
    wr = [None, None]
    gp = issue_gather(units[0][1], units[0][2], 0)
    for u, (plane, src_hbm, d) in enumerate(units):
        b = u & 1
        cur = gp
        if u + 1 < len(units):
            nxt = units[u + 1]
            gp = issue_gather(nxt[1], nxt[2], 1 - b)
        for dsc in cur:
            dsc.wait()
        if wr[b] is not None:
            wr[b].wait()

        def tr_body(s, carry, b=b):
            soff = (s >> 7) * 1024 + (s & (GB - 1))
            plsc.store_scatter(tb[b], [tvec0 + soff],
                               rows[b][s, pl.ds(0, L)])
            plsc.store_scatter(tb[b], [tvec1 + soff],
                               rows[b][s, pl.ds(L, L)])
            return carry

        lax.fori_loop(0, NN, tr_body, 0)
        wr[b] = pltpu.async_copy(tb[b], out_hbm.at[plane, dst0 + d],
                                 osem[b])
    wr[0].wait()
    wr[1].wait()


_sc_scatter = functools.partial(
    pl.kernel,
    out_type=jax.ShapeDtypeStruct((2, NN, OPS * NN), jnp.float32),
    mesh=plsc.VectorSubcoreMesh(core_axis_name="c", subcore_axis_name="s"),
    compiler_params=pltpu.CompilerParams(
        needs_layout_passes=False, use_tc_tiling_on_sc=False,
        disable_bounds_checks=True),
    scratch_types=[
        pltpu.VMEM((S // GB, GB), jnp.int32),
        pltpu.VMEM((CH,), jnp.int32),
        pltpu.VMEM((CH,), jnp.int32),
        pltpu.VMEM((CH,), jnp.int32),
        pltpu.VMEM((CH,), jnp.int32),
        pltpu.VMEM((NN, OPS), jnp.float32),
        pltpu.VMEM((NN, OPS), jnp.float32),
        pltpu.VMEM((OPS * NN,), jnp.float32),
        pltpu.VMEM((OPS * NN,), jnp.float32),
        pltpu.SemaphoreType.DMA,
        pltpu.SemaphoreType.DMA,
        pltpu.SemaphoreType.DMA,
        pltpu.SemaphoreType.DMA,
        pltpu.SemaphoreType.DMA,
        pltpu.SemaphoreType.DMA,
    ],
)(_sc_body)


def kernel(alphas, noise_u, idx):
    idx = idx.astype(jnp.int32)
    w_t, a_t = _sample(alphas.T, noise_u.T)
    w_t4 = w_t.reshape(4, 8, EP // GB, GB).transpose(0, 2, 1, 3)
    a_t4 = a_t.reshape(4, 8, EP // GB, GB).transpose(0, 2, 1, 3)
    w_e, a_e = _sc_transpose(w_t4, a_t4)
    out = _sc_scatter(idx[0], idx[1],
                      w_e.reshape(EP, OPS), a_e.reshape(EP, OPS))
    out6 = out.reshape(2, NN, 4, 4, 8, GB)
    return out6.transpose(0, 1, 3, 5, 2, 4).reshape(2, NN, NN, OPS)


# speedup vs baseline: 1.0523x; 1.0171x over previous
import functools

import jax
import jax.numpy as jnp
from jax import lax
from jax.experimental import pallas as pl
from jax.experimental.pallas import tpu as pltpu
from jax.experimental.pallas import tpu_sc as plsc

NN = 512
OPS = 32
E = NN * NN // 2
NSLOT = NN * NN
PAD = 16384
EP = E + PAD
NC, NS, L = 2, 16, 16
NW = NC * NS
S = NSLOT // NW
DPW = S // NN
CH = 8192
GB = 128
TB = 2048
EPT = EP // NW
CH2 = EPT // 12


def _tc_sample_body(a_ref, u_ref, i_ref, w_ref, l_ref, lin_ref):
    i = pl.program_id(0)
    a = a_ref[...]
    u = jnp.clip(u_ref[...], 1e-6, 1.0 - 1e-6)
    w = u / (u + (1.0 - u) * jnp.exp(-a))
    is_pad = i >= E // TB
    w_ref[...] = jnp.where(is_pad, 0.0, w)
    l_ref[...] = jnp.where(is_pad, 0.0, a)
    lin_ref[...] = i_ref[0:1, :] * NN + i_ref[1:2, :]


def _sample(a_t, u_t, idx):
    last = E // TB - 1
    return pl.pallas_call(
        _tc_sample_body,
        grid=(EP // TB,),
        in_specs=[pl.BlockSpec((OPS, TB), lambda i: (0, jnp.minimum(i, last)))] * 2
        + [pl.BlockSpec((2, TB), lambda i: (0, jnp.minimum(i, last)))],
        out_specs=[pl.BlockSpec((OPS, TB), lambda i: (0, i))] * 2
        + [pl.BlockSpec((1, TB), lambda i: (0, i))],
        out_shape=[jax.ShapeDtypeStruct((OPS, EP), jnp.float32)] * 2
        + [jax.ShapeDtypeStruct((1, EP), jnp.int32)],
    )(a_t, u_t, idx)


def _sc_transpose_body(wt_hbm, at_hbm, we_hbm, ae_hbm,
                       tin0, tin1, tout0, tout1,
                       isem0, isem1, osem0, osem1):
    wid = lax.axis_index("s") * NC + lax.axis_index("c")
    iota = lax.broadcasted_iota(jnp.int32, (L,), 0)
    iota32 = iota * OPS
    ccw = CH2 // GB
    tin = (tin0, tin1)
    tout = (tout0, tout1)
    isem = (isem0, isem1)
    osem = (osem0, osem1)

    units = [(src, dst, c)
             for src, dst in ((wt_hbm, we_hbm), (at_hbm, ae_hbm))
             for c in range(EPT // CH2)]

    def issue_load(u, b):
        src, _, c = units[u]
        e0 = wid * EPT + c * CH2
        return pltpu.async_copy(src.at[:, pl.ds(e0 // GB, ccw), :, :],
                                tin[b], isem[b])

    wr = [None, None]
    pend = issue_load(0, 0)
    for u in range(len(units)):
        b = u & 1
        cur = pend
        if u + 1 < len(units):
            pend = issue_load(u + 1, 1 - b)
        cur.wait()
        if wr[b] is not None:
            wr[b].wait()

        def tr_body(i, carry, b=b):
            cc = i >> 3
            ec = i & 7
            for rb in range(4):
                for r_in in range(8):
                    v = tin[b][rb, cc, r_in, pl.ds(ec * L, L)]
                    plsc.store_scatter(
                        tout[b],
                        [iota32 + ((cc * GB + ec * L) * OPS
                                   + rb * 8 + r_in)], v)
            return carry

        lax.fori_loop(0, CH2 // L, tr_body, 0)
        _, dst, c = units[u]
        e0 = wid * EPT + c * CH2
        wr[b] = pltpu.async_copy(tout[b], dst.at[pl.ds(e0 * OPS, CH2 * OPS)],
                                 osem[b])
    wr[0].wait()
    wr[1].wait()


_sc_transpose = functools.partial(
    pl.kernel,
    out_type=(jax.ShapeDtypeStruct((EP * OPS,), jnp.float32),
              jax.ShapeDtypeStruct((EP * OPS,), jnp.float32)),
    mesh=plsc.VectorSubcoreMesh(core_axis_name="c", subcore_axis_name="s"),
    compiler_params=pltpu.CompilerParams(
        needs_layout_passes=False, use_tc_tiling_on_sc=False,
        disable_bounds_checks=True),
    scratch_types=[
        pltpu.VMEM((4, CH2 // GB, 8, GB), jnp.float32),
        pltpu.VMEM((4, CH2 // GB, 8, GB), jnp.float32),
        pltpu.VMEM((CH2 * OPS,), jnp.float32),
        pltpu.VMEM((CH2 * OPS,), jnp.float32),
        pltpu.SemaphoreType.DMA,
        pltpu.SemaphoreType.DMA,
        pltpu.SemaphoreType.DMA,
        pltpu.SemaphoreType.DMA,
    ],
)(_sc_transpose_body)


def _sc_body(lin_hbm, opw_hbm, alph_hbm, out_hbm,
             win, lb0, lb1, rows0, rows1, rows2, tb0, tb1,
             csem0, csem1, gsem0, gsem1, gsem2, osem0, osem1):
    wid = lax.axis_index("s") * NC + lax.axis_index("c")
    base = wid * S
    dst0 = wid * DPW
    iota = lax.broadcasted_iota(jnp.int32, (L,), 0)

    def init_row(j, carry):
        for k in range(GB // L):
            sent = E + ((j * GB + k * L + iota) & (PAD - 1))
            win[j, pl.ds(k * L, L)] = sent
        return carry

    lax.fori_loop(0, S // GB, init_row, 0)

    lb = (lb0, lb1)
    csem = (csem0, csem1)
    UNROLL = 8
    NCHK = E // CH

    def issue_chunk(c, b):
        return pltpu.async_copy(lin_hbm.at[pl.ds(c * CH, CH)], lb[b], csem[b])

    pend = issue_chunk(0, 0)
    for c in range(NCHK):
        b = c & 1
        cur = pend
        if c + 1 < NCHK:
            pend = issue_chunk(c + 1, 1 - b)
        cur.wait()

        def scan_body(i, carry, c=c, b=b):
            for k in range(UNROLL):
                off = i * (UNROLL * L) + k * L
                rel = lb[b][pl.ds(off, L)] - base
                m = (rel >= 0) & (rel < S)
                relc = jnp.where(m, rel, 0)
                evec = (c * CH) + off + iota
                plsc.store_scatter(
                    win, [relc >> 7, relc & (GB - 1)], evec, mask=m)
            return carry

        lax.fori_loop(0, CH // (UNROLL * L), scan_body, 0)

    rows = (rows0, rows1, rows2)
    tb = (tb0, tb1)
    gsem = (gsem0, gsem1, gsem2)
    osem = (osem0, osem1)
    tvec0 = (iota >> 3) * (4 * 1024) + (iota & 7) * GB
    tvec1 = ((iota + L) >> 3) * (4 * 1024) + ((iota + L) & 7) * GB

    def issue_gather(src_hbm, d, b):
        return tuple(
            pltpu.async_copy(src_hbm.at[win.at[d * 4 + q]],
                             rows[b].at[pl.ds(q * GB, GB)], gsem[b])
            for q in range(4))

    units = [(p, s, d) for p, s in ((0, opw_hbm), (1, alph_hbm))
             for d in range(DPW)]
    wr = [None, None]
    gp = [None, None, None]
    gp[0] = issue_gather(units[0][1], units[0][2], 0)
    gp[1] = issue_gather(units[1][1], units[1][2], 1)
    for u, (plane, src_hbm, d) in enumerate(units):
        gb = u % 3
        if u + 2 < len(units):
            nxt = units[u + 2]
            gp[(u + 2) % 3] = issue_gather(nxt[1], nxt[2], (u + 2) % 3)
        for dsc in gp[gb]:
            dsc.wait()
        wb = u & 1
        if wr[wb] is not None:
            wr[wb].wait()

        def tr_body(s, carry, gb=gb, wb=wb):
            soff = (s >> 7) * 1024 + (s & (GB - 1))
            plsc.store_scatter(tb[wb], [tvec0 + soff],
                               rows[gb][s, pl.ds(0, L)])
            plsc.store_scatter(tb[wb], [tvec1 + soff],
                               rows[gb][s, pl.ds(L, L)])
            return carry

        lax.fori_loop(0, NN, tr_body, 0)
        wr[wb] = pltpu.async_copy(tb[wb], out_hbm.at[plane, dst0 + d],
                                  osem[wb])
    wr[0].wait()
    wr[1].wait()


_sc_scatter = functools.partial(
    pl.kernel,
    out_type=jax.ShapeDtypeStruct((2, NN, OPS * NN), jnp.float32),
    mesh=plsc.VectorSubcoreMesh(core_axis_name="c", subcore_axis_name="s"),
    compiler_params=pltpu.CompilerParams(
        needs_layout_passes=False, use_tc_tiling_on_sc=False,
        disable_bounds_checks=True),
    scratch_types=[
        pltpu.VMEM((S // GB, GB), jnp.int32),
        pltpu.VMEM((CH,), jnp.int32),
        pltpu.VMEM((CH,), jnp.int32),
        pltpu.VMEM((NN, OPS), jnp.float32),
        pltpu.VMEM((NN, OPS), jnp.float32),
        pltpu.VMEM((NN, OPS), jnp.float32),
        pltpu.VMEM((OPS * NN,), jnp.float32),
        pltpu.VMEM((OPS * NN,), jnp.float32),
        pltpu.SemaphoreType.DMA,
        pltpu.SemaphoreType.DMA,
        pltpu.SemaphoreType.DMA,
        pltpu.SemaphoreType.DMA,
        pltpu.SemaphoreType.DMA,
        pltpu.SemaphoreType.DMA,
        pltpu.SemaphoreType.DMA,
    ],
)(_sc_body)


def kernel(alphas, noise_u, idx):
    idx = idx.astype(jnp.int32)
    w_t, a_t, lin = _sample(alphas.T, noise_u.T, idx)
    w_t4 = w_t.reshape(4, 8, EP // GB, GB).transpose(0, 2, 1, 3)
    a_t4 = a_t.reshape(4, 8, EP // GB, GB).transpose(0, 2, 1, 3)
    w_e, a_e = _sc_transpose(w_t4, a_t4)
    out = _sc_scatter(lin.reshape(EP),
                      w_e.reshape(EP, OPS), a_e.reshape(EP, OPS))
    out6 = out.reshape(2, NN, 4, 4, 8, GB)
    return out6.transpose(0, 1, 3, 5, 2, 4).reshape(2, NN, NN, OPS)
